# baseline (speedup 1.0000x reference)
import functools

import jax
import jax.numpy as jnp
from jax import lax
from jax.experimental import pallas as pl
from jax.experimental.pallas import tpu as pltpu
from jax.experimental.pallas import tpu_sc as plsc

_NUM_TYPES = 87
_F = 128
_N = 100000
_CH = 128
_BW = 256


def _fuse_body(extra_ref, onehot_ref, econf_ref, wconf_ref, w1_ref, out_ref):
    cfg = lax.dot_general(
        econf_ref[...], wconf_ref[...], (((1,), (1,)), ((), ())),
        preferred_element_type=jnp.float32)
    cat = jnp.concatenate([extra_ref[...], onehot_ref[...], cfg], axis=1)
    out_ref[...] = lax.dot_general(
        cat, w1_ref[...], (((1,), (1,)), ((), ())),
        preferred_element_type=jnp.float32)


def _fused_table(extra, onehot, econf, wconf, w1):
    return pl.pallas_call(
        _fuse_body,
        out_shape=jax.ShapeDtypeStruct((_NUM_TYPES, _F), jnp.float32),
    )(extra, onehot, econf, wconf, w1)


@functools.cache
def _make_gather():
    info = plsc.get_sparse_core_info()
    nc, ns = info.num_cores, info.num_subcores
    nw = nc * ns
    b_per_w = ((_N + nw - 1) // nw + _CH - 1) // _CH * _CH
    n_chunks = b_per_w // _CH

    mesh = plsc.VectorSubcoreMesh(core_axis_name="c", subcore_axis_name="s")

    @functools.partial(
        pl.kernel,
        out_type=(jax.ShapeDtypeStruct((_N, _F), jnp.float32),
                  jax.ShapeDtypeStruct((_N, _F), jnp.float32)),
        mesh=mesh,
        scratch_types=[
            pltpu.VMEM((b_per_w,), jnp.int32),
            pltpu.VMEM((_BW, _F), jnp.float32),
            pltpu.VMEM((_BW, _F), jnp.float32),
            pltpu.VMEM_SHARED((_NUM_TYPES, _F), jnp.float32),
            pltpu.SemaphoreType.DMA,
            pltpu.SemaphoreType.DMA,
            pltpu.SemaphoreType.DMA,
            pltpu.SemaphoreType.DMA,
            pltpu.SemaphoreType.DMA,
        ],
    )
    def gather_k(idx_hbm, table_hbm, out_hbm, out2_hbm, idx_v,
                 buf_a, buf_b, tab_s,
                 isem, gsem_a, gsem_b, ssem_a, ssem_b):
        sid = lax.axis_index("s")
        wid = sid * nc + lax.axis_index("c")
        start = wid * b_per_w

        @pl.when(sid == 0)
        def _():
            pltpu.sync_copy(table_hbm, tab_s)
        plsc.subcore_barrier()

        rows = [_BW] * (b_per_w // _BW) + (
            [b_per_w % _BW] if b_per_w % _BW else [])
        n_win = len(rows)
        woff = [sum(rows[:p]) for p in range(n_win)]
        s = [pl.multiple_of(jnp.minimum(start + woff[p], _N - rows[p]), 32)
             for p in range(n_win)]

        ih = [pltpu.async_copy(idx_hbm.at[pl.ds(s[p] + j * _CH, _CH)],
                               idx_v.at[pl.ds(woff[p] + j * _CH, _CH)], isem)
              for p in range(n_win) for j in range(rows[p] // _CH)]
        for h in ih:
            h.wait()

        nbuf = 2
        bufs = (buf_a, buf_b)
        gsems = (gsem_a, gsem_b)
        ssems = (ssem_a, ssem_b)
        gh = [[] for _ in range(n_win)]
        sh = [None] * n_win
        sh2 = [None] * n_win

        def issue_stores(p):
            b = p % nbuf
            src = bufs[b].at[pl.ds(0, rows[p])] if rows[p] != _BW else bufs[b]
            sh[p] = pltpu.async_copy(
                src, out_hbm.at[pl.ds(s[p], rows[p])], ssems[b])
            sh2[p] = pltpu.async_copy(
                src, out2_hbm.at[pl.ds(s[p], rows[p])], ssems[b])

        for p in range(n_win):
            b = p % nbuf
            if p >= nbuf:
                sh[p - nbuf].wait()
                sh2[p - nbuf].wait()
            for j in range(rows[p] // _CH):
                gh[p].append(pltpu.async_copy(
                    tab_s.at[idx_v.at[pl.ds(woff[p] + j * _CH, _CH)]],
                    bufs[b].at[pl.ds(j * _CH, _CH)], gsems[b]))
            if p >= 1:
                for h in gh[p - 1]:
                    h.wait()
                issue_stores(p - 1)
        last = n_win - 1
        for h in gh[last]:
            h.wait()
        issue_stores(last)
        for p in range(max(0, n_win - nbuf), n_win):
            sh[p].wait()
            sh2[p].wait()

    return gather_k


def kernel(atom_types, extra_table, W_onehot, electron_config, W_config, W1):
    table = _fused_table(extra_table, W_onehot, electron_config, W_config, W1)
    out, out2 = _make_gather()(atom_types.astype(jnp.int32), table)
    return out, out2


# speedup vs baseline: 1.0077x; 1.0077x over previous
import functools

import jax
import jax.numpy as jnp
from jax import lax
from jax.experimental import pallas as pl
from jax.experimental.pallas import tpu as pltpu
from jax.experimental.pallas import tpu_sc as plsc

_NUM_TYPES = 87
_F = 128
_N = 100000
_CH = 128
_BW = 256


def _fuse_body(extra_ref, onehot_ref, econf_ref, wconf_ref, w1_ref, out_ref):
    cfg = lax.dot_general(
        econf_ref[...], wconf_ref[...], (((1,), (1,)), ((), ())),
        preferred_element_type=jnp.float32)
    cat = jnp.concatenate([extra_ref[...], onehot_ref[...], cfg], axis=1)
    out_ref[...] = lax.dot_general(
        cat, w1_ref[...], (((1,), (1,)), ((), ())),
        preferred_element_type=jnp.float32)


def _fused_table(extra, onehot, econf, wconf, w1):
    return pl.pallas_call(
        _fuse_body,
        out_shape=jax.ShapeDtypeStruct((_NUM_TYPES, _F), jnp.float32),
    )(extra, onehot, econf, wconf, w1)


@functools.cache
def _make_gather():
    info = plsc.get_sparse_core_info()
    nc, ns = info.num_cores, info.num_subcores
    nw = nc * ns
    b_per_w = ((_N + nw - 1) // nw + _CH - 1) // _CH * _CH
    n_chunks = b_per_w // _CH

    mesh = plsc.VectorSubcoreMesh(core_axis_name="c", subcore_axis_name="s")

    @functools.partial(
        pl.kernel,
        out_type=(jax.ShapeDtypeStruct((_N, _F), jnp.float32),
                  jax.ShapeDtypeStruct((_N, _F), jnp.float32)),
        mesh=mesh,
        scratch_types=[
            pltpu.VMEM((b_per_w,), jnp.int32),
            pltpu.VMEM((_BW, _F), jnp.float32),
            pltpu.VMEM((_BW, _F), jnp.float32),
            pltpu.VMEM_SHARED((_NUM_TYPES, _F), jnp.float32),
            pltpu.SemaphoreType.DMA,
            pltpu.SemaphoreType.DMA,
            pltpu.SemaphoreType.DMA,
            pltpu.SemaphoreType.DMA,
            pltpu.SemaphoreType.DMA,
        ],
    )
    def gather_k(idx_hbm, table_hbm, out_hbm, out2_hbm, idx_v,
                 buf_a, buf_b, tab_s,
                 isem, gsem_a, gsem_b, ssem_a, ssem_b):
        sid = lax.axis_index("s")
        wid = sid * nc + lax.axis_index("c")
        start = wid * b_per_w

        rows = [_BW] * (b_per_w // _BW) + (
            [b_per_w % _BW] if b_per_w % _BW else [])
        n_win = len(rows)
        woff = [sum(rows[:p]) for p in range(n_win)]
        s = [pl.multiple_of(jnp.minimum(start + woff[p], _N - rows[p]), 32)
             for p in range(n_win)]

        ih = [[pltpu.async_copy(idx_hbm.at[pl.ds(s[p] + j * _CH, _CH)],
                                idx_v.at[pl.ds(woff[p] + j * _CH, _CH)], isem)
               for j in range(rows[p] // _CH)] for p in range(n_win)]

        @pl.when(sid == 0)
        def _():
            pltpu.sync_copy(table_hbm, tab_s)
        plsc.subcore_barrier()

        nbuf = 2
        bufs = (buf_a, buf_b)
        gsems = (gsem_a, gsem_b)
        ssems = (ssem_a, ssem_b)
        gh = [[] for _ in range(n_win)]
        sh = [None] * n_win
        sh2 = [None] * n_win

        def issue_stores(p):
            b = p % nbuf
            src = bufs[b].at[pl.ds(0, rows[p])] if rows[p] != _BW else bufs[b]
            sh[p] = pltpu.async_copy(
                src, out_hbm.at[pl.ds(s[p], rows[p])], ssems[b])
            sh2[p] = pltpu.async_copy(
                src, out2_hbm.at[pl.ds(s[p], rows[p])], ssems[b])

        for p in range(n_win):
            b = p % nbuf
            if p >= nbuf:
                sh[p - nbuf].wait()
                sh2[p - nbuf].wait()
            for h in ih[p]:
                h.wait()
            for j in range(rows[p] // _CH):
                gh[p].append(pltpu.async_copy(
                    tab_s.at[idx_v.at[pl.ds(woff[p] + j * _CH, _CH)]],
                    bufs[b].at[pl.ds(j * _CH, _CH)], gsems[b]))
            if p >= 1:
                for h in gh[p - 1]:
                    h.wait()
                issue_stores(p - 1)
        last = n_win - 1
        for h in gh[last]:
            h.wait()
        issue_stores(last)
        for p in range(max(0, n_win - nbuf), n_win):
            sh[p].wait()
            sh2[p].wait()

    return gather_k


def kernel(atom_types, extra_table, W_onehot, electron_config, W_config, W1):
    table = _fused_table(extra_table, W_onehot, electron_config, W_config, W1)
    out, out2 = _make_gather()(atom_types.astype(jnp.int32), table)
    return out, out2


# speedup vs baseline: 1.0094x; 1.0016x over previous
import functools

import jax
import jax.numpy as jnp
from jax import lax
from jax.experimental import pallas as pl
from jax.experimental.pallas import tpu as pltpu
from jax.experimental.pallas import tpu_sc as plsc

_NUM_TYPES = 87
_F = 128
_N = 100000
_CH = 128
_BW = 256


def _fuse_body(extra_ref, onehot_ref, econf_ref, wconf_ref, w1_ref, out_ref):
    cfg = lax.dot_general(
        econf_ref[...], wconf_ref[...], (((1,), (1,)), ((), ())),
        preferred_element_type=jnp.float32)
    cat = jnp.concatenate([extra_ref[...], onehot_ref[...], cfg], axis=1)
    out_ref[...] = lax.dot_general(
        cat, w1_ref[...], (((1,), (1,)), ((), ())),
        preferred_element_type=jnp.float32)


def _fused_table(extra, onehot, econf, wconf, w1):
    return pl.pallas_call(
        _fuse_body,
        out_shape=jax.ShapeDtypeStruct((_NUM_TYPES, _F), jnp.float32),
    )(extra, onehot, econf, wconf, w1)


@functools.cache
def _make_gather():
    info = plsc.get_sparse_core_info()
    nc, ns = info.num_cores, info.num_subcores
    nw = nc * ns
    b_per_w = ((_N + nw - 1) // nw + _CH - 1) // _CH * _CH

    mesh = plsc.VectorSubcoreMesh(core_axis_name="c", subcore_axis_name="s")

    @functools.partial(
        pl.kernel,
        out_type=(jax.ShapeDtypeStruct((_N, _F), jnp.float32),
                  jax.ShapeDtypeStruct((_N, _F), jnp.float32)),
        mesh=mesh,
        scratch_types=[
            pltpu.VMEM((b_per_w,), jnp.int32),
            pltpu.VMEM((_BW, _F), jnp.float32),
            pltpu.VMEM((_BW, _F), jnp.float32),
            pltpu.VMEM_SHARED((_NUM_TYPES, _F), jnp.float32),
            pltpu.SemaphoreType.DMA,
            pltpu.SemaphoreType.DMA,
            pltpu.SemaphoreType.DMA,
            pltpu.SemaphoreType.DMA,
            pltpu.SemaphoreType.DMA,
        ],
    )
    def gather_k(idx_hbm, table_hbm, out_hbm, out2_hbm, idx_v,
                 buf_a, buf_b, tab_s,
                 isem, gsem_a, gsem_b, ssem_a, ssem_b):
        sid = lax.axis_index("s")
        wid = sid * nc + lax.axis_index("c")
        start = wid * b_per_w

        rows = [_BW] * (b_per_w // _BW) + (
            [b_per_w % _BW] if b_per_w % _BW else [])
        n_win = len(rows)
        woff = [sum(rows[:p]) for p in range(n_win)]
        s = [pl.multiple_of(jnp.minimum(start + woff[p], _N - rows[p]), 32)
             for p in range(n_win)]

        ih = [[pltpu.async_copy(idx_hbm.at[pl.ds(s[p] + j * _CH, _CH)],
                                idx_v.at[pl.ds(woff[p] + j * _CH, _CH)], isem)
               for j in range(rows[p] // _CH)] for p in range(n_win)]

        @pl.when(sid == 0)
        def _():
            pltpu.sync_copy(table_hbm, tab_s)
        plsc.subcore_barrier()

        nbuf = 2
        bufs = (buf_a, buf_b)
        gsems = (gsem_a, gsem_b)
        ssems = (ssem_a, ssem_b)
        gh = [[] for _ in range(n_win)]
        sh = [None] * n_win
        sh2 = [None] * n_win

        def issue_stores(p):
            b = p % nbuf
            src = bufs[b].at[pl.ds(0, rows[p])] if rows[p] != _BW else bufs[b]
            sh[p] = pltpu.async_copy(
                src, out_hbm.at[pl.ds(s[p], rows[p])], ssems[b])
            sh2[p] = pltpu.async_copy(
                src, out2_hbm.at[pl.ds(s[p], rows[p])], ssems[b])

        for p in range(n_win):
            b = p % nbuf
            if p >= nbuf:
                sh[p - nbuf].wait()
                sh2[p - nbuf].wait()
            for h in ih[p]:
                h.wait()
            for j in range(rows[p] // _CH):
                gh[p].append(pltpu.async_copy(
                    tab_s.at[idx_v.at[pl.ds(woff[p] + j * _CH, _CH)]],
                    bufs[b].at[pl.ds(j * _CH, _CH)], gsems[b]))
            if p >= 1:
                for h in gh[p - 1]:
                    h.wait()
                issue_stores(p - 1)
        last = n_win - 1
        for h in gh[last]:
            h.wait()
        issue_stores(last)
        for p in range(max(0, n_win - nbuf), n_win):
            sh[p].wait()
            sh2[p].wait()

    return gather_k


def kernel(atom_types, extra_table, W_onehot, electron_config, W_config, W1):
    table = _fused_table(extra_table, W_onehot, electron_config, W_config, W1)
    out, out2 = _make_gather()(atom_types.astype(jnp.int32), table)
    return out, out2
